# Initial kernel scaffold; baseline (speedup 1.0000x reference)
import jax
import jax.numpy as jnp
from jax.experimental import pallas as pl


def kernel(x, fc_weight, bias):
    raise NotImplementedError("write your pallas kernel here")


# speedup vs baseline: 1.2440x; 1.2440x over previous
import functools

import jax
import jax.numpy as jnp
from jax import lax
from jax.experimental import pallas as pl
from jax.experimental.pallas import tpu as pltpu
from jax.experimental.pallas import tpu_sc as plsc

NUM_FIELDS = 26
FIELD_DIM = 38461
TOTAL_ROWS = NUM_FIELDS * FIELD_DIM
BATCH = 16384
LANES = 16
NUM_CORES = 2
NUM_SUBCORES = 16
SC_BATCH = BATCH // NUM_CORES
TILE_BATCH = SC_BATCH // NUM_SUBCORES
VECS_PER_TILE = TILE_BATCH // LANES
VECS_PER_FIELD = SC_BATCH // LANES
TBL_LEN = FIELD_DIM + 11
PAD_ROWS = 16


def _body(x_hbm, w_hbm, b_hbm, out_hbm,
          x_v, xt_v, tbl_v, idx_v, part_v, red_v, out_v, bias_v,
          xt_sh, part_sh):
    c = lax.axis_index("c")
    s = lax.axis_index("s")
    gbase = c * SC_BATCH + s * TILE_BATCH
    lanes = lax.broadcasted_iota(jnp.int32, (LANES,), 0)

    pltpu.sync_copy(x_hbm.at[pl.ds(gbase * NUM_FIELDS, TILE_BATCH * NUM_FIELDS)],
                    x_v)

    def t_body(k, carry):
        row = (k * LANES + lanes) * NUM_FIELDS
        for f in range(NUM_FIELDS):
            xt_v[f, pl.ds(k * LANES, LANES)] = plsc.load_gather(x_v, [row + f])
        return carry

    lax.fori_loop(0, VECS_PER_TILE, t_body, 0)
    for f in range(NUM_FIELDS):
        pltpu.sync_copy(xt_v.at[f, :],
                        xt_sh.at[f, pl.ds(s * TILE_BATCH, TILE_BATCH)])
    plsc.subcore_barrier()

    def do_field(f):
        start = f * FIELD_DIM
        start8 = pl.multiple_of((start // 8) * 8, 8)
        adj = start - start8
        pltpu.sync_copy(w_hbm.at[pl.ds(start8, TBL_LEN)], tbl_v)
        pltpu.sync_copy(xt_sh.at[f, :], idx_v)

        def g_body(k, carry):
            iv = idx_v[pl.ds(k * LANES, LANES)] + adj
            part_v[pl.ds(k * LANES, LANES)] = plsc.load_gather(tbl_v, [iv])
            return carry

        lax.fori_loop(0, VECS_PER_FIELD, g_body, 0)
        pltpu.sync_copy(part_v, part_sh.at[f, :])

    do_field(s)

    @pl.when(s + NUM_SUBCORES < NUM_FIELDS)
    def _():
        do_field(s + NUM_SUBCORES)

    plsc.subcore_barrier()

    pltpu.sync_copy(b_hbm, bias_v)
    for f in range(NUM_FIELDS):
        pltpu.sync_copy(part_sh.at[f, pl.ds(s * TILE_BATCH, TILE_BATCH)],
                        red_v.at[f, :])
    bias_vec = bias_v[...]

    def r_body(k, carry):
        acc = red_v[0, pl.ds(k * LANES, LANES)]
        for f in range(1, NUM_FIELDS):
            acc = acc + red_v[f, pl.ds(k * LANES, LANES)]
        out_v[pl.ds(k * LANES, LANES)] = acc + bias_vec
        return carry

    lax.fori_loop(0, VECS_PER_TILE, r_body, 0)
    pltpu.sync_copy(out_v, out_hbm.at[pl.ds(gbase, TILE_BATCH)])


@jax.jit
def _features_linear(x, w_pad, b16):
    mesh = plsc.VectorSubcoreMesh(core_axis_name="c", subcore_axis_name="s")
    return pl.kernel(
        _body,
        out_type=jax.ShapeDtypeStruct((BATCH,), jnp.float32),
        mesh=mesh,
        compiler_params=pltpu.CompilerParams(
            needs_layout_passes=False, use_tc_tiling_on_sc=False),
        scratch_types=[
            pltpu.VMEM((TILE_BATCH * NUM_FIELDS,), jnp.int32),
            pltpu.VMEM((NUM_FIELDS, TILE_BATCH), jnp.int32),
            pltpu.VMEM((TBL_LEN,), jnp.float32),
            pltpu.VMEM((SC_BATCH,), jnp.int32),
            pltpu.VMEM((SC_BATCH,), jnp.float32),
            pltpu.VMEM((NUM_FIELDS, TILE_BATCH), jnp.float32),
            pltpu.VMEM((TILE_BATCH,), jnp.float32),
            pltpu.VMEM((LANES,), jnp.float32),
            pltpu.VMEM_SHARED((NUM_FIELDS, SC_BATCH), jnp.int32),
            pltpu.VMEM_SHARED((NUM_FIELDS, SC_BATCH), jnp.float32),
        ],
    )(x, w_pad, b16)


def kernel(x, fc_weight, bias):
    w_pad = jnp.concatenate(
        [fc_weight.reshape(-1), jnp.zeros((PAD_ROWS,), jnp.float32)])
    b16 = jnp.broadcast_to(bias.astype(jnp.float32), (LANES,))
    out = _features_linear(x.reshape(-1), w_pad, b16)
    return out.reshape(BATCH, 1)
